# baseline (speedup 1.0000x reference)
import functools

import jax
import jax.numpy as jnp
from jax import lax
from jax.experimental import pallas as pl
from jax.experimental.pallas import tpu as pltpu
from jax.experimental.pallas import tpu_sc as plsc

VOCAB = 100000
H = 768
POS = 4096
B = 4
S = 4096

NW = 32
ROWS = B * S
RPW = ROWS // NW
CH = 64
NBUF = 2
NCH = RPW // CH
SCALE = float(H) ** 0.5
EPS = 1e-5

TR = 2048
NTB = ROWS // TR
SB = S // TR

_mesh = plsc.VectorSubcoreMesh(core_axis_name="c", subcore_axis_name="s")


@functools.partial(
    pl.kernel,
    out_type=jax.ShapeDtypeStruct((ROWS, H), jnp.float32),
    mesh=_mesh,
    compiler_params=pltpu.CompilerParams(needs_layout_passes=False),
    scratch_types=[
        pltpu.VMEM((RPW,), jnp.int32),
        pltpu.VMEM((NBUF, CH, H), jnp.float32),
        [pltpu.SemaphoreType.DMA] * NBUF,
    ],
)
def _gather_kernel(ids_hbm, ww_hbm, out_hbm, idx_v, rows_v, sems):
    wid = lax.axis_index("s") * 2 + lax.axis_index("c")
    base = wid * RPW
    pltpu.sync_copy(ids_hbm.at[pl.ds(base, RPW)], idx_v)
    copies = [None] * NBUF
    for p in range(NBUF):
        copies[p] = pltpu.async_copy(
            ww_hbm.at[idx_v.at[pl.ds(p * CH, CH)]], rows_v.at[p], sems[p])
    for c in range(NCH):
        b = c % NBUF
        copies[b].wait()
        pltpu.sync_copy(rows_v.at[b], out_hbm.at[pl.ds(base + c * CH, CH)])
        if c + NBUF < NCH:
            copies[b] = pltpu.async_copy(
                ww_hbm.at[idx_v.at[pl.ds((c + NBUF) * CH, CH)]],
                rows_v.at[b], sems[b])


def _ln_body(g_ref, p_ref, s_ref, ws_ref, ga_ref, be_ref, o_ref):
    x = g_ref[...] * SCALE + p_ref[...]
    sidf = s_ref[0, 0, :].astype(jnp.float32)[:, None]
    x = x + ws_ref[0:1, :] + sidf * (ws_ref[1:2, :] - ws_ref[0:1, :])
    mu = jnp.mean(x, axis=-1, keepdims=True)
    var = jnp.mean(x * x, axis=-1, keepdims=True) - mu * mu
    o_ref[...] = (x - mu) * lax.rsqrt(var + EPS) * ga_ref[...] + be_ref[...]


_ln_call = pl.pallas_call(
    _ln_body,
    grid=(NTB,),
    in_specs=[
        pl.BlockSpec((TR, H), lambda i: ((i % B) * SB + i // B, 0)),
        pl.BlockSpec((TR, H), lambda i: (i // B, 0)),
        pl.BlockSpec((1, 1, TR), lambda i: ((i % B) * SB + i // B, 0, 0)),
        pl.BlockSpec((2, H), lambda i: (0, 0)),
        pl.BlockSpec((1, H), lambda i: (0, 0)),
        pl.BlockSpec((1, H), lambda i: (0, 0)),
    ],
    out_specs=pl.BlockSpec((TR, H), lambda i: ((i % B) * SB + i // B, 0)),
    out_shape=jax.ShapeDtypeStruct((ROWS, H), jnp.float32),
)


def kernel(input_ids, segment_ids, W_word, W_seg, gamma, beta, pos_enc):
    ids = input_ids.reshape(ROWS).astype(jnp.int32)
    seg3 = segment_ids.reshape(NTB, 1, TR).astype(jnp.int32)
    gathered = _gather_kernel(ids, W_word)
    out = _ln_call(gathered, pos_enc, seg3, W_seg,
                   gamma.reshape(1, H), beta.reshape(1, H))
    return out.reshape(B, S, H)


# speedup vs baseline: 1.0053x; 1.0053x over previous
import functools

import jax
import jax.numpy as jnp
from jax import lax
from jax.experimental import pallas as pl
from jax.experimental.pallas import tpu as pltpu
from jax.experimental.pallas import tpu_sc as plsc

VOCAB = 100000
H = 768
POS = 4096
B = 4
S = 4096

NW = 32
ROWS = B * S
RPW = ROWS // NW
CH = 32
NBUF = 4
NCH = RPW // CH
SCALE = float(H) ** 0.5
EPS = 1e-5

TR = 2048
NTB = ROWS // TR
SB = S // TR

_mesh = plsc.VectorSubcoreMesh(core_axis_name="c", subcore_axis_name="s")


@functools.partial(
    pl.kernel,
    out_type=jax.ShapeDtypeStruct((ROWS, H), jnp.float32),
    mesh=_mesh,
    compiler_params=pltpu.CompilerParams(needs_layout_passes=False),
    scratch_types=[
        pltpu.VMEM((RPW,), jnp.int32),
        pltpu.VMEM((NBUF, CH, H), jnp.float32),
        [pltpu.SemaphoreType.DMA] * NBUF,
    ],
)
def _gather_kernel(ids_hbm, ww_hbm, out_hbm, idx_v, rows_v, sems):
    wid = lax.axis_index("s") * 2 + lax.axis_index("c")
    base = wid * RPW
    pltpu.sync_copy(ids_hbm.at[pl.ds(base, RPW)], idx_v)
    copies = [None] * NBUF
    for p in range(NBUF):
        copies[p] = pltpu.async_copy(
            ww_hbm.at[idx_v.at[pl.ds(p * CH, CH)]], rows_v.at[p], sems[p])
    for c in range(NCH):
        b = c % NBUF
        copies[b].wait()
        pltpu.sync_copy(rows_v.at[b], out_hbm.at[pl.ds(base + c * CH, CH)])
        if c + NBUF < NCH:
            copies[b] = pltpu.async_copy(
                ww_hbm.at[idx_v.at[pl.ds((c + NBUF) * CH, CH)]],
                rows_v.at[b], sems[b])


def _ln_body(g_ref, p_ref, s_ref, ws_ref, ga_ref, be_ref, o_ref):
    x = g_ref[...] * SCALE + p_ref[...]
    sidf = s_ref[0, 0, :].astype(jnp.float32)[:, None]
    x = x + ws_ref[0:1, :] + sidf * (ws_ref[1:2, :] - ws_ref[0:1, :])
    mu = jnp.mean(x, axis=-1, keepdims=True)
    var = jnp.mean(x * x, axis=-1, keepdims=True) - mu * mu
    o_ref[...] = (x - mu) * lax.rsqrt(var + EPS) * ga_ref[...] + be_ref[...]


_ln_call = pl.pallas_call(
    _ln_body,
    grid=(NTB,),
    in_specs=[
        pl.BlockSpec((TR, H), lambda i: ((i % B) * SB + i // B, 0)),
        pl.BlockSpec((TR, H), lambda i: (i // B, 0)),
        pl.BlockSpec((1, 1, TR), lambda i: ((i % B) * SB + i // B, 0, 0)),
        pl.BlockSpec((2, H), lambda i: (0, 0)),
        pl.BlockSpec((1, H), lambda i: (0, 0)),
        pl.BlockSpec((1, H), lambda i: (0, 0)),
    ],
    out_specs=pl.BlockSpec((TR, H), lambda i: ((i % B) * SB + i // B, 0)),
    out_shape=jax.ShapeDtypeStruct((ROWS, H), jnp.float32),
)


def kernel(input_ids, segment_ids, W_word, W_seg, gamma, beta, pos_enc):
    ids = input_ids.reshape(ROWS).astype(jnp.int32)
    seg3 = segment_ids.reshape(NTB, 1, TR).astype(jnp.int32)
    gathered = _gather_kernel(ids, W_word)
    out = _ln_call(gathered, pos_enc, seg3, W_seg,
                   gamma.reshape(1, H), beta.reshape(1, H))
    return out.reshape(B, S, H)
